# baseline (speedup 1.0000x reference)
import functools

import jax
import jax.numpy as jnp
from jax import lax
from jax.experimental import pallas as pl
from jax.experimental.pallas import tpu as pltpu
from jax.experimental.pallas import tpu_sc as plsc

PRE_SEQ_LEN = 128
LAYER_NUM = 32
HEAD_NUM_KV = 8
SIZE_PER_HEAD = 128
EMB_DIM = LAYER_NUM * SIZE_PER_HEAD * HEAD_NUM_KV * 2
BATCH = 8

CHUNKS = EMB_DIM // SIZE_PER_HEAD
NUM_TASKS = BATCH * CHUNKS
NUM_WORKERS = 32
CH_PER_TILE = CHUNKS // NUM_WORKERS
LANES = 16

CPP = 2
PHASES = CH_PER_TILE // CPP
TPP = BATCH * CPP
NBUF = 4
ROUNDS = TPP // NBUF

_mesh = plsc.VectorSubcoreMesh(core_axis_name="core", subcore_axis_name="subcore")


@functools.partial(
    pl.kernel,
    out_type=jax.ShapeDtypeStruct((NUM_TASKS, PRE_SEQ_LEN, SIZE_PER_HEAD), jnp.float32),
    mesh=_mesh,
    scratch_types=[
        pltpu.VMEM((BATCH, PRE_SEQ_LEN), jnp.int32),
        pltpu.VMEM((PRE_SEQ_LEN, CPP, SIZE_PER_HEAD), jnp.float32),
        pltpu.VMEM((NBUF, PRE_SEQ_LEN, SIZE_PER_HEAD), jnp.float32),
        pltpu.SemaphoreType.DMA,
    ]
    + [pltpu.SemaphoreType.DMA] * NBUF,
)
def _gather_kernel(table, prefix, out, pfx_v, stage_v, obuf_v, ssem, *osem):
    wid = lax.axis_index("subcore") * 2 + lax.axis_index("core")
    ch0 = wid * CH_PER_TILE

    pltpu.sync_copy(prefix, pfx_v)

    def fire_out(buf, t):
        pltpu.async_copy(obuf_v.at[buf], out.at[t], osem[buf])

    def wait_out(buf, t):
        pltpu.make_async_copy(obuf_v.at[buf], out.at[t], osem[buf]).wait()

    def phase_body(p, carry):
        cstart = ch0 + p * CPP
        pltpu.async_copy(table.at[:, pl.ds(cstart, CPP)], stage_v, ssem)
        pltpu.make_async_copy(table.at[:, pl.ds(cstart, CPP)], stage_v, ssem).wait()

        def round_body(r, carry2):
            for bb in range(NBUF):
                k = r * NBUF + bb
                j = p * TPP + k
                b = k // CPP
                cl = k % CPP
                t = b * CHUNKS + cstart + cl

                @pl.when(j >= NBUF)
                def _():
                    wait_out(bb, t)

                def group_body(g, carry3):
                    s0 = g * LANES
                    pvec = pfx_v[b, pl.ds(s0, LANES)]
                    for i in range(LANES):
                        row = pvec[i]
                        for kk in range(SIZE_PER_HEAD // LANES):
                            sl = pl.ds(kk * LANES, LANES)
                            obuf_v[bb, s0 + i, sl] = stage_v[row, cl, sl]
                    return carry3

                lax.fori_loop(0, PRE_SEQ_LEN // LANES, group_body, 0)
                fire_out(bb, t)
            return carry2

        lax.fori_loop(0, ROUNDS, round_body, 0)
        return carry

    lax.fori_loop(0, PHASES, phase_body, 0)

    for bb in range(NBUF):
        wait_out(bb, 0)


def kernel(prefix, embedding):
    table = embedding.reshape(PRE_SEQ_LEN, CHUNKS, SIZE_PER_HEAD)
    out = _gather_kernel(table, prefix)
    return out.reshape(BATCH, LAYER_NUM * 2, HEAD_NUM_KV, PRE_SEQ_LEN, SIZE_PER_HEAD)


# speedup vs baseline: 3.1623x; 3.1623x over previous
import functools

import jax
import jax.numpy as jnp
from jax import lax
from jax.experimental import pallas as pl
from jax.experimental.pallas import tpu as pltpu
from jax.experimental.pallas import tpu_sc as plsc

PRE_SEQ_LEN = 128
LAYER_NUM = 32
HEAD_NUM_KV = 8
SIZE_PER_HEAD = 128
EMB_DIM = LAYER_NUM * SIZE_PER_HEAD * HEAD_NUM_KV * 2
BATCH = 8

CHUNKS = EMB_DIM // SIZE_PER_HEAD
NUM_TASKS = BATCH * CHUNKS
LANES = 16

PCH = 16
PHASES = (CHUNKS // 2) // PCH
TPP = PCH // 2
NBUF = 4
LAG = 3
NSTAGE = 3
R_PER_TILE = PRE_SEQ_LEN // 16

_mesh = plsc.VectorSubcoreMesh(core_axis_name="core", subcore_axis_name="subcore")


@functools.partial(
    pl.kernel,
    out_type=jax.ShapeDtypeStruct((NUM_TASKS, PRE_SEQ_LEN, SIZE_PER_HEAD), jnp.float32),
    mesh=_mesh,
    scratch_types=[
        pltpu.VMEM((PRE_SEQ_LEN,), jnp.int32),
        pltpu.VMEM((NBUF, PRE_SEQ_LEN), jnp.int32),
        pltpu.VMEM((NBUF, PRE_SEQ_LEN, SIZE_PER_HEAD), jnp.float32),
        pltpu.VMEM_SHARED((NSTAGE, PRE_SEQ_LEN * PCH, SIZE_PER_HEAD), jnp.float32),
        pltpu.SemaphoreType.DMA,
    ]
    + [pltpu.SemaphoreType.DMA] * (2 * NBUF),
)
def _gather_kernel(table, prefix, out, base_v, idx_v, rows_v, stage, ssem, *sems):
    gsem = sems[:NBUF]
    osem = sems[NBUF:]
    core = lax.axis_index("core")
    sid = lax.axis_index("subcore")
    b = sid // 2
    half = sid % 2
    c_sc = core * (CHUNKS // 2)

    pltpu.sync_copy(prefix.at[b], base_v)
    for i in range(PRE_SEQ_LEN // LANES):
        sl = pl.ds(i * LANES, LANES)
        base_v[sl] = base_v[sl] * PCH

    def fire_staging(p):
        nb = p % NSTAGE
        cstart = c_sc + p * PCH
        for k in range(R_PER_TILE):
            r = sid * R_PER_TILE + k
            pltpu.async_copy(
                table.at[r, pl.ds(cstart, PCH)], stage.at[nb, pl.ds(r * PCH, PCH)], ssem
            )

    def wait_staging(p):
        nb = p % NSTAGE
        cstart = c_sc + p * PCH
        for k in range(R_PER_TILE):
            r = sid * R_PER_TILE + k
            pltpu.make_async_copy(
                table.at[r, pl.ds(cstart, PCH)], stage.at[nb, pl.ds(r * PCH, PCH)], ssem
            ).wait()

    def out_row(p, k):
        return b * CHUNKS + c_sc + p * PCH + half * TPP + k

    def fire_out(buf, t):
        pltpu.async_copy(rows_v.at[buf], out.at[t], osem[buf])

    def wait_out(buf, t):
        pltpu.make_async_copy(rows_v.at[buf], out.at[t], osem[buf]).wait()

    def fire_gather(p, k):
        bb = k % NBUF
        j = p * TPP + k
        cl = half * TPP + k

        @pl.when(j >= NBUF)
        def _():
            wait_out(bb, out_row(p, k))

        for i in range(PRE_SEQ_LEN // LANES):
            sl = pl.ds(i * LANES, LANES)
            idx_v[bb, sl] = base_v[sl] + cl
        pltpu.async_copy(stage.at[p % NSTAGE].at[idx_v.at[bb]], rows_v.at[bb], gsem[bb])

    def retire(p, k):
        bb = k % NBUF
        pltpu.make_async_copy(
            stage.at[p % NSTAGE].at[idx_v.at[bb]], rows_v.at[bb], gsem[bb]
        ).wait()
        fire_out(bb, out_row(p, k))

    fire_staging(0)
    wait_staging(0)
    plsc.subcore_barrier()

    def phase_body(p, carry):
        @pl.when(p < PHASES - 1)
        def _():
            fire_staging(p + 1)

        for l in range(LAG):
            fire_gather(p, l)

            @pl.when(p > 0)
            def _():
                retire(p - 1, TPP - LAG + l)

        for k in range(TPP - LAG):
            fire_gather(p, k + LAG)
            retire(p, k)

        @pl.when(p < PHASES - 1)
        def _():
            wait_staging(p + 1)

        plsc.subcore_barrier()
        return carry

    lax.fori_loop(0, PHASES, phase_body, 0)

    for l in range(LAG):
        retire(PHASES - 1, TPP - LAG + l)
    for k in range(TPP - NBUF, TPP):
        wait_out(k % NBUF, out_row(PHASES - 1, k))


def kernel(prefix, embedding):
    table = embedding.reshape(PRE_SEQ_LEN, CHUNKS, SIZE_PER_HEAD)
    out = _gather_kernel(table, prefix)
    return out.reshape(BATCH, LAYER_NUM * 2, HEAD_NUM_KV, PRE_SEQ_LEN, SIZE_PER_HEAD)
